# baseline (speedup 1.0000x reference)
import functools

import jax
import jax.numpy as jnp
from jax import lax
from jax.experimental import pallas as pl
from jax.experimental.pallas import tpu as pltpu
from jax.experimental.pallas import tpu_sc as plsc

_N = 10000
_NP = 10240
_B = 2
_EPAD = 163840
_BLK = 128
_HID = 32


@functools.lru_cache(maxsize=None)
def _make_prop(nc):
    mesh = plsc.VectorSubcoreMesh(core_axis_name="c", subcore_axis_name="s")
    ept = _EPAD // 16
    nblk = ept // _BLK
    npt = _NP // 16
    cc = 128

    @functools.partial(
        pl.kernel, mesh=mesh,
        out_type=jax.ShapeDtypeStruct((_NP, 2 * nc * cc), jnp.float32),
        scratch_types=[
            pltpu.VMEM((_BLK,), jnp.int32),
            pltpu.VMEM((_BLK,), jnp.int32),
            pltpu.VMEM((_BLK,), jnp.int32),
            pltpu.VMEM((_BLK,), jnp.int32),
            pltpu.VMEM((_BLK // 8, 128), jnp.float32),
            pltpu.VMEM((_BLK // 8, 128), jnp.float32),
            pltpu.VMEM((_BLK, cc), jnp.float32),
            pltpu.VMEM((_BLK, cc), jnp.float32),
            pltpu.VMEM_SHARED((_NP, cc), jnp.float32),
            pltpu.SemaphoreType.DMA,
            pltpu.SemaphoreType.DMA,
            pltpu.SemaphoreType.DMA,
            pltpu.SemaphoreType.DMA,
            pltpu.SemaphoreType.DMA,
            pltpu.SemaphoreType.DMA,
        ],
    )
    def prop(vh, z, rowh, colh, wh, oh,
             rv0, rv1, cv0, cv1, wv0, wv1, gb0, gb1, acc,
             si0, si1, sg0, sg1, ss0, ss1):
        c = lax.axis_index("c")
        s = lax.axis_index("s")
        rstripe = pl.ds(s * npt, npt)
        bufs = ((rv0, cv0, wv0, gb0, si0, sg0, ss0),
                (rv1, cv1, wv1, gb1, si1, sg1, ss1))

        def idx_start(i, h):
            rv, cv, wv = bufs[h][0], bufs[h][1], bufs[h][2]
            si = bufs[h][4]
            base = pl.ds(s * ept + i * _BLK, _BLK)
            pltpu.async_copy(rowh.at[base], rv, si)
            pltpu.async_copy(colh.at[base], cv, si)
            woff = pl.multiple_of((s * ept + i * _BLK) // 8, 8)
            pltpu.async_copy(wh.at[pl.ds(woff, _BLK // 8), :], wv, si)

        def idx_drain(h):
            rv, cv, wv = bufs[h][0], bufs[h][1], bufs[h][2]
            si = bufs[h][4]
            hb = pl.ds(0, _BLK)
            pltpu.make_async_copy(rowh.at[hb], rv, si).wait()
            pltpu.make_async_copy(colh.at[hb], cv, si).wait()
            pltpu.make_async_copy(wh.at[pl.ds(0, _BLK // 8), :], wv, si).wait()

        for k in range(nc):
            off = pl.multiple_of((c * nc + k) * cc, cc)
            csl = pl.ds(off, cc)
            pltpu.sync_copy(z.at[rstripe], acc.at[rstripe])
            plsc.subcore_barrier()

            idx_start(0, 0)
            idx_start(1, 1)
            idx_drain(0)
            pltpu.async_copy(vh.at[rv0, csl], gb0, sg0)

            def body(i2, carry):
                for h in range(2):
                    rv, cv, wv, gb, si, sg, ss = bufs[h]
                    o = bufs[1 - h]
                    i = i2 * 2 + h
                    pltpu.make_async_copy(vh.at[rv, csl], gb, sg).wait()

                    @pl.when(i + 1 < nblk)
                    def _():
                        idx_drain(1 - h)

                        @pl.when(i >= 1)
                        def _():
                            pltpu.make_async_copy(
                                z.at[pl.ds(0, _BLK)], o[3], o[6]).wait()
                        pltpu.async_copy(vh.at[o[0], csl], o[3], o[5])

                    def sbody(g, cy):
                        for el in range(8):
                            wvec = wv[g, pl.ds(el * 16, 16)]
                            e = g * 8 + el
                            for j in range(cc // 16):
                                sl = pl.ds(j * 16, 16)
                                gb[e, sl] = gb[e, sl] * wvec
                        return cy
                    lax.fori_loop(0, _BLK // 8, sbody, 0, unroll=2)

                    pltpu.async_copy(gb, acc.at[cv], ss, add=True)

                    @pl.when(i + 2 < nblk)
                    def _():
                        idx_start(i + 2, h)
                return carry

            lax.fori_loop(0, nblk // 2, body, 0)
            hb = pl.ds(0, _BLK)
            pltpu.make_async_copy(z.at[hb], gb0, ss0).wait()
            pltpu.make_async_copy(z.at[hb], gb1, ss1).wait()
            plsc.subcore_barrier()
            pltpu.sync_copy(acc.at[rstripe], oh.at[rstripe, csl])
            plsc.subcore_barrier()

    return prop


@functools.lru_cache(maxsize=None)
def _make_nw():
    mesh = plsc.VectorSubcoreMesh(core_axis_name="c", subcore_axis_name="s")
    epw = _EPAD // 32

    @functools.partial(
        pl.kernel, mesh=mesh,
        out_type=jax.ShapeDtypeStruct((_EPAD,), jnp.float32),
        compiler_params=pltpu.CompilerParams(needs_layout_passes=False),
        scratch_types=[
            pltpu.VMEM((_NP,), jnp.float32),
            pltpu.VMEM((epw,), jnp.int32),
            pltpu.VMEM((epw,), jnp.int32),
            pltpu.VMEM((epw,), jnp.float32),
            pltpu.VMEM((epw,), jnp.float32),
        ],
    )
    def nwk(dish, rowh, colh, ewh, nwh, disv, rowv, colv, ewv, nwv):
        c = lax.axis_index("c")
        s = lax.axis_index("s")
        base = (s * 2 + c) * epw
        pltpu.sync_copy(dish, disv)
        pltpu.sync_copy(rowh.at[pl.ds(base, epw)], rowv)
        pltpu.sync_copy(colh.at[pl.ds(base, epw)], colv)
        pltpu.sync_copy(ewh.at[pl.ds(base, epw)], ewv)

        def body(i, cy):
            sl = pl.ds(i * 16, 16)
            r16 = rowv[sl]
            c16 = colv[sl]
            e16 = ewv[sl]
            dr = plsc.load_gather(disv, [r16])
            dc = plsc.load_gather(disv, [c16])
            v = -(dr * e16 * dc)
            v = jnp.where(r16 == c16, jnp.zeros_like(v), v)
            nwv[sl] = v
            return cy

        lax.fori_loop(0, epw // 16, body, 0)
        pltpu.sync_copy(nwv, nwh.at[pl.ds(base, epw)])

    return nwk


def _prop_all(V, rowp, colp, w):
    C = V.shape[1]
    nc = -(-C // 256)
    cpad = nc * 256
    if cpad != C:
        V = jnp.concatenate([V, jnp.zeros((_NP, cpad - C), jnp.float32)], axis=1)
    z = jnp.zeros((_NP, 128), jnp.float32)
    w16 = jnp.broadcast_to(w[:, None], (_EPAD, 16)).reshape(_EPAD // 8, 128)
    out = _make_prop(nc)(V, z, rowp, colp, w16)
    return out[:, :C] if cpad != C else out


def _dot(a, b):
    return lax.dot_general(a, b, (((1,), (0,)), ((), ())),
                           preferred_element_type=jnp.float32)


@functools.lru_cache(maxsize=None)
def _make_tconv(T, cin, cout, nblk):
    T1 = T - 2

    def body(x_ref, w_ref, b_ref, o_ref):
        w = w_ref[...]
        bb = b_ref[...]
        for t in range(T1):
            xs = x_ref[0, :, pl.ds(t * cin, 3 * cin)]
            h = _dot(xs, w) + bb
            p = h[:, :cout]
            q = h[:, cout:2 * cout]
            r = h[:, 2 * cout:]
            o_ref[0, :, pl.ds(t * cout, cout)] = jnp.maximum(
                p * jax.nn.sigmoid(q) + r, 0.0)

    return pl.pallas_call(
        body,
        compiler_params=pltpu.CompilerParams(vmem_limit_bytes=100 * 1024 * 1024),
        grid=(_NP // nblk, _B),
        in_specs=[
            pl.BlockSpec((1, nblk, T * cin), lambda i, b: (b, i, 0)),
            pl.BlockSpec((3 * cin, 3 * cout), lambda i, b: (0, 0)),
            pl.BlockSpec((1, 3 * cout), lambda i, b: (0, 0)),
        ],
        out_specs=pl.BlockSpec((1, nblk, T1 * cout), lambda i, b: (b, i, 0)),
        out_shape=jax.ShapeDtypeStruct((_B, _NP, T1 * cout), jnp.float32),
    )


@functools.lru_cache(maxsize=None)
def _make_cheb_combine(M, mb):
    def body(t0_ref, s1_ref, s2_ref, w_ref, b_ref, o_ref):
        t0 = t0_ref[...]
        s1 = s1_ref[...]
        s2 = s2_ref[...]
        w = w_ref[...]
        y = (_dot(t0, w[0:32]) + _dot(s1, w[32:64])
             + _dot(2.0 * s2 - t0, w[64:96]) + b_ref[...])
        o_ref[...] = jnp.maximum(y, 0.0)

    return pl.pallas_call(
        body,
        compiler_params=pltpu.CompilerParams(vmem_limit_bytes=100 * 1024 * 1024),
        grid=(M // mb,),
        in_specs=[
            pl.BlockSpec((mb, 32), lambda i: (i, 0)),
            pl.BlockSpec((mb, 32), lambda i: (i, 0)),
            pl.BlockSpec((mb, 32), lambda i: (i, 0)),
            pl.BlockSpec((96, 32), lambda i: (0, 0)),
            pl.BlockSpec((1, 32), lambda i: (0, 0)),
        ],
        out_specs=pl.BlockSpec((mb, 32), lambda i: (i, 0)),
        out_shape=jax.ShapeDtypeStruct((M, 32), jnp.float32),
    )


@functools.lru_cache(maxsize=None)
def _make_tconv_bn(T, nblk, final):
    T2 = T - 2
    cm = _HID
    cnt = float(_B * T2 * cm)

    def body(u_ref, w_ref, b_ref, g_ref, bt_ref, lw_ref, lb_ref, o_ref, scr):
        w = w_ref[...]
        bb = b_ref[...]
        ssum = None
        ssq = None
        for b in range(_B):
            for t in range(T2):
                xs = u_ref[b, :, pl.ds(t * cm, 3 * cm)]
                h = _dot(xs, w) + bb
                p = h[:, :cm]
                q = h[:, cm:2 * cm]
                r = h[:, 2 * cm:]
                y = jnp.maximum(p * jax.nn.sigmoid(q) + r, 0.0)
                scr[b, :, pl.ds(t * cm, cm)] = y
                s1 = jnp.sum(y, axis=1, keepdims=True)
                s2 = jnp.sum(y * y, axis=1, keepdims=True)
                ssum = s1 if ssum is None else ssum + s1
                ssq = s2 if ssq is None else ssq + s2
        mean = ssum / cnt
        var = ssq / cnt - mean * mean
        inv = lax.rsqrt(var + 1e-5)
        gam = g_ref[...]
        bet = bt_ref[...]
        for b in range(_B):
            for t in range(T2):
                y = (scr[b, :, pl.ds(t * cm, cm)] - mean) * inv * gam + bet
                y = jnp.maximum(y, 0.0)
                if final:
                    y = _dot(y, lw_ref[...]) + lb_ref[...]
                    o_ref[b, :, pl.ds(t, 1)] = y
                else:
                    o_ref[b, :, pl.ds(t * cm, cm)] = y

    return pl.pallas_call(
        body,
        compiler_params=pltpu.CompilerParams(vmem_limit_bytes=100 * 1024 * 1024),
        grid=(_NP // nblk,),
        in_specs=[
            pl.BlockSpec((_B, nblk, T * cm), lambda i: (0, i, 0)),
            pl.BlockSpec((3 * cm, 3 * cm), lambda i: (0, 0)),
            pl.BlockSpec((1, 3 * cm), lambda i: (0, 0)),
            pl.BlockSpec((nblk, 1), lambda i: (i, 0)),
            pl.BlockSpec((nblk, 1), lambda i: (i, 0)),
            pl.BlockSpec((cm, 1), lambda i: (0, 0)),
            pl.BlockSpec((1, 1), lambda i: (0, 0)),
        ],
        out_specs=pl.BlockSpec((_B, nblk, T2 if final else T2 * cm),
                               lambda i: (0, i, 0)),
        out_shape=jax.ShapeDtypeStruct(
            (_B, _NP, T2 if final else T2 * cm), jnp.float32),
        scratch_shapes=[pltpu.VMEM((_B, nblk, T2 * cm), jnp.float32)],
    )


def _tc_weights(p, cin, cout):
    ws = []
    bs = []
    for i in range(3):
        w = p['w%d' % (i + 1)]
        ws.append(jnp.transpose(w[:, :, 0, :], (2, 1, 0)).reshape(3 * cin, cout))
        bs.append(p['b%d' % (i + 1)])
    return jnp.concatenate(ws, axis=1), jnp.concatenate(bs).reshape(1, 3 * cout)


def _stconv(h, rowp, colp, nwp, p, cin, final, lin_w, lin_b):
    T = h.shape[2] // cin
    T1 = T - 2
    w1, b1 = _tc_weights(p['tc1'], cin, _HID)
    t0 = _make_tconv(T, cin, _HID, 1024)(h, w1, b1)
    J = _B * T1
    V0 = jnp.transpose(t0, (1, 0, 2)).reshape(_NP, J * _HID)
    s1 = _prop_all(V0, rowp, colp, nwp)
    s2 = _prop_all(s1, rowp, colp, nwp)
    M = _NP * J
    cheb_w = p['cheb_w'].reshape(3 * _HID, _HID)
    cheb_b = p['cheb_b'].reshape(1, _HID)
    g = _make_cheb_combine(M, 2048)(
        V0.reshape(M, _HID), s1.reshape(M, _HID), s2.reshape(M, _HID),
        cheb_w, cheb_b)
    u = jnp.transpose(g.reshape(_NP, _B, T1 * _HID), (1, 0, 2))
    w2, b2 = _tc_weights(p['tc2'], _HID, _HID)
    npad = _NP - _N
    g2 = jnp.concatenate([p['bn_g'], jnp.ones((npad,), jnp.float32)]).reshape(_NP, 1)
    bt2 = jnp.concatenate([p['bn_b'], jnp.zeros((npad,), jnp.float32)]).reshape(_NP, 1)
    return _make_tconv_bn(T1, 1024, final)(u, w2, b2, g2, bt2, lin_w, lin_b)


def kernel(x, edge_index, edge_weight, params):
    row = edge_index[0]
    col = edge_index[1]
    ew = jnp.where(row == col, 0.0, edge_weight)
    pad = _EPAD - row.shape[0]
    zi = jnp.zeros((pad,), jnp.int32)
    rowp = jnp.concatenate([row, zi])
    colp = jnp.concatenate([col, zi])
    ewp = jnp.concatenate([ew, jnp.zeros((pad,), jnp.float32)])

    ones = jnp.ones((_NP, 256), jnp.float32)
    dega = _make_prop(1)(ones, jnp.zeros((_NP, 128), jnp.float32),
                         rowp, rowp,
                         jnp.broadcast_to(ewp[:, None], (_EPAD, 16))
                         .reshape(_EPAD // 8, 128))
    deg = dega[:, 0]
    dis = jnp.where(deg > 0, lax.rsqrt(jnp.where(deg > 0, deg, 1.0)), 0.0)
    nwp = _make_nw()(dis, rowp, colp, ewp)

    B, T, _, cin = x.shape
    xn = jnp.transpose(x, (0, 2, 1, 3)).reshape(B, _N, T * cin)
    xn = jnp.concatenate(
        [xn, jnp.zeros((B, _NP - _N, T * cin), jnp.float32)], axis=1)
    lin_w = params['lin_w'].T
    lin_b = params['lin_b'].reshape(1, 1)
    h = _stconv(xn, rowp, colp, nwp, params['stconv1'], 2, False, lin_w, lin_b)
    h = _stconv(h, rowp, colp, nwp, params['stconv2'], _HID, True, lin_w, lin_b)
    return jnp.transpose(h[:, :_N, :], (0, 2, 1))[..., None]


# speedup vs baseline: 1.0746x; 1.0746x over previous
import functools

import jax
import jax.numpy as jnp
from jax import lax
from jax.experimental import pallas as pl
from jax.experimental.pallas import tpu as pltpu
from jax.experimental.pallas import tpu_sc as plsc

_N = 10000
_NP = 10240
_B = 2
_EPAD = 163840
_BLK = 64
_HID = 32


@functools.lru_cache(maxsize=None)
def _make_prop(nc):
    mesh = plsc.VectorSubcoreMesh(core_axis_name="c", subcore_axis_name="s")
    ept = _EPAD // 16
    nblk = ept // _BLK
    npt = _NP // 16
    cc = 128
    ns = 4

    @functools.partial(
        pl.kernel, mesh=mesh,
        out_type=jax.ShapeDtypeStruct((_NP, 2 * nc * cc), jnp.float32),
        scratch_types=(
            [pltpu.VMEM((_BLK,), jnp.int32) for _ in range(ns)]
            + [pltpu.VMEM((_BLK,), jnp.int32) for _ in range(ns)]
            + [pltpu.VMEM((_BLK // 8, 128), jnp.float32) for _ in range(ns)]
            + [pltpu.VMEM((_BLK, cc), jnp.float32) for _ in range(ns)]
            + [pltpu.VMEM_SHARED((_NP, cc), jnp.float32)]
            + [pltpu.SemaphoreType.DMA for _ in range(4 * ns)]
        ),
    )
    def prop(vh, z, rowh, colh, wh, oh, *bufs):
        rv = bufs[0:ns]
        cv = bufs[ns:2 * ns]
        wv = bufs[2 * ns:3 * ns]
        gb = bufs[3 * ns:4 * ns]
        acc = bufs[4 * ns]
        sr = bufs[4 * ns + 1:4 * ns + 1 + ns]
        scw = bufs[4 * ns + 1 + ns:4 * ns + 1 + 2 * ns]
        sg = bufs[4 * ns + 1 + 2 * ns:4 * ns + 1 + 3 * ns]
        ss = bufs[4 * ns + 1 + 3 * ns:4 * ns + 1 + 4 * ns]
        c = lax.axis_index("c")
        s = lax.axis_index("s")
        rstripe = pl.ds(s * npt, npt)
        hb = pl.ds(0, _BLK)
        hw = pl.ds(0, _BLK // 8)

        def row_start(i, h):
            pltpu.async_copy(rowh.at[pl.ds(s * ept + i * _BLK, _BLK)],
                             rv[h], sr[h])

        def row_drain(h):
            pltpu.make_async_copy(rowh.at[hb], rv[h], sr[h]).wait()

        def colw_start(i, h):
            pltpu.async_copy(colh.at[pl.ds(s * ept + i * _BLK, _BLK)],
                             cv[h], scw[h])
            woff = pl.multiple_of((s * ept + i * _BLK) // 8, 8)
            pltpu.async_copy(wh.at[pl.ds(woff, _BLK // 8), :], wv[h], scw[h])

        def colw_drain(h):
            pltpu.make_async_copy(colh.at[hb], cv[h], scw[h]).wait()
            pltpu.make_async_copy(wh.at[hw, :], wv[h], scw[h]).wait()

        def scat_drain(h):
            pltpu.make_async_copy(z.at[hb], gb[h], ss[h]).wait()

        for k in range(nc):
            off = pl.multiple_of((c * nc + k) * cc, cc)
            csl = pl.ds(off, cc)
            pltpu.sync_copy(z.at[rstripe], acc.at[rstripe])
            plsc.subcore_barrier()

            for j in range(ns):
                row_start(j, j)
            for j in range(2):
                colw_start(j, j)
            for j in range(2):
                row_drain(j)
                pltpu.async_copy(vh.at[rv[j], csl], gb[j], sg[j])

            def body(i4, carry):
                for h in range(ns):
                    h2 = (h + 2) % ns
                    i = i4 * ns + h
                    pltpu.make_async_copy(vh.at[rv[h], csl], gb[h], sg[h]).wait()

                    @pl.when(i + 4 < nblk)
                    def _():
                        row_start(i + 4, h)

                    @pl.when(i + 2 < nblk)
                    def _():
                        @pl.when(i >= 2)
                        def _():
                            scat_drain(h2)
                        row_drain(h2)
                        pltpu.async_copy(vh.at[rv[h2], csl], gb[h2], sg[h2])
                        colw_start(i + 2, h2)

                    colw_drain(h)
                    def sbody(g, cy):
                        for el in range(8):
                            wvec = wv[h][g, pl.ds(el * 16, 16)]
                            e = g * 8 + el
                            for j in range(cc // 16):
                                sl = pl.ds(j * 16, 16)
                                gb[h][e, sl] = gb[h][e, sl] * wvec
                        return cy
                    lax.fori_loop(0, _BLK // 8, sbody, 0, unroll=2)

                    pltpu.async_copy(gb[h], acc.at[cv[h]], ss[h], add=True)
                return carry

            lax.fori_loop(0, nblk // ns, body, 0)
            scat_drain((nblk - 2) % ns)
            scat_drain((nblk - 1) % ns)
            plsc.subcore_barrier()
            pltpu.sync_copy(acc.at[rstripe], oh.at[rstripe, csl])
            plsc.subcore_barrier()

    return prop


@functools.lru_cache(maxsize=None)
def _make_nw():
    mesh = plsc.VectorSubcoreMesh(core_axis_name="c", subcore_axis_name="s")
    epw = _EPAD // 32

    @functools.partial(
        pl.kernel, mesh=mesh,
        out_type=jax.ShapeDtypeStruct((_EPAD,), jnp.float32),
        compiler_params=pltpu.CompilerParams(needs_layout_passes=False),
        scratch_types=[
            pltpu.VMEM((_NP,), jnp.float32),
            pltpu.VMEM((epw,), jnp.int32),
            pltpu.VMEM((epw,), jnp.int32),
            pltpu.VMEM((epw,), jnp.float32),
            pltpu.VMEM((epw,), jnp.float32),
        ],
    )
    def nwk(dish, rowh, colh, ewh, nwh, disv, rowv, colv, ewv, nwv):
        c = lax.axis_index("c")
        s = lax.axis_index("s")
        base = (s * 2 + c) * epw
        pltpu.sync_copy(dish, disv)
        pltpu.sync_copy(rowh.at[pl.ds(base, epw)], rowv)
        pltpu.sync_copy(colh.at[pl.ds(base, epw)], colv)
        pltpu.sync_copy(ewh.at[pl.ds(base, epw)], ewv)

        def body(i, cy):
            sl = pl.ds(i * 16, 16)
            r16 = rowv[sl]
            c16 = colv[sl]
            e16 = ewv[sl]
            dr = plsc.load_gather(disv, [r16])
            dc = plsc.load_gather(disv, [c16])
            v = -(dr * e16 * dc)
            v = jnp.where(r16 == c16, jnp.zeros_like(v), v)
            nwv[sl] = v
            return cy

        lax.fori_loop(0, epw // 16, body, 0)
        pltpu.sync_copy(nwv, nwh.at[pl.ds(base, epw)])

    return nwk


def _prop_all(V, rowp, colp, w):
    C = V.shape[1]
    nc = -(-C // 256)
    cpad = nc * 256
    if cpad != C:
        V = jnp.concatenate([V, jnp.zeros((_NP, cpad - C), jnp.float32)], axis=1)
    z = jnp.zeros((_NP, 128), jnp.float32)
    w16 = jnp.broadcast_to(w[:, None], (_EPAD, 16)).reshape(_EPAD // 8, 128)
    out = _make_prop(nc)(V, z, rowp, colp, w16)
    return out[:, :C] if cpad != C else out


def _dot(a, b):
    return lax.dot_general(a, b, (((1,), (0,)), ((), ())),
                           preferred_element_type=jnp.float32)


@functools.lru_cache(maxsize=None)
def _make_tconv(T, cin, cout, nblk):
    T1 = T - 2

    def body(x_ref, w_ref, b_ref, o_ref):
        w = w_ref[...]
        bb = b_ref[...]
        for t in range(T1):
            xs = x_ref[0, :, pl.ds(t * cin, 3 * cin)]
            h = _dot(xs, w) + bb
            p = h[:, :cout]
            q = h[:, cout:2 * cout]
            r = h[:, 2 * cout:]
            o_ref[0, :, pl.ds(t * cout, cout)] = jnp.maximum(
                p * jax.nn.sigmoid(q) + r, 0.0)

    return pl.pallas_call(
        body,
        compiler_params=pltpu.CompilerParams(vmem_limit_bytes=100 * 1024 * 1024),
        grid=(_NP // nblk, _B),
        in_specs=[
            pl.BlockSpec((1, nblk, T * cin), lambda i, b: (b, i, 0)),
            pl.BlockSpec((3 * cin, 3 * cout), lambda i, b: (0, 0)),
            pl.BlockSpec((1, 3 * cout), lambda i, b: (0, 0)),
        ],
        out_specs=pl.BlockSpec((1, nblk, T1 * cout), lambda i, b: (b, i, 0)),
        out_shape=jax.ShapeDtypeStruct((_B, _NP, T1 * cout), jnp.float32),
    )


@functools.lru_cache(maxsize=None)
def _make_cheb_combine(M, mb):
    def body(t0_ref, s1_ref, s2_ref, w_ref, b_ref, o_ref):
        t0 = t0_ref[...]
        s1 = s1_ref[...]
        s2 = s2_ref[...]
        w = w_ref[...]
        y = (_dot(t0, w[0:32]) + _dot(s1, w[32:64])
             + _dot(2.0 * s2 - t0, w[64:96]) + b_ref[...])
        o_ref[...] = jnp.maximum(y, 0.0)

    return pl.pallas_call(
        body,
        compiler_params=pltpu.CompilerParams(vmem_limit_bytes=100 * 1024 * 1024),
        grid=(M // mb,),
        in_specs=[
            pl.BlockSpec((mb, 32), lambda i: (i, 0)),
            pl.BlockSpec((mb, 32), lambda i: (i, 0)),
            pl.BlockSpec((mb, 32), lambda i: (i, 0)),
            pl.BlockSpec((96, 32), lambda i: (0, 0)),
            pl.BlockSpec((1, 32), lambda i: (0, 0)),
        ],
        out_specs=pl.BlockSpec((mb, 32), lambda i: (i, 0)),
        out_shape=jax.ShapeDtypeStruct((M, 32), jnp.float32),
    )


@functools.lru_cache(maxsize=None)
def _make_tconv_bn(T, nblk, final):
    T2 = T - 2
    cm = _HID
    cnt = float(_B * T2 * cm)

    def body(u_ref, w_ref, b_ref, g_ref, bt_ref, lw_ref, lb_ref, o_ref, scr):
        w = w_ref[...]
        bb = b_ref[...]
        ssum = None
        ssq = None
        for b in range(_B):
            for t in range(T2):
                xs = u_ref[b, :, pl.ds(t * cm, 3 * cm)]
                h = _dot(xs, w) + bb
                p = h[:, :cm]
                q = h[:, cm:2 * cm]
                r = h[:, 2 * cm:]
                y = jnp.maximum(p * jax.nn.sigmoid(q) + r, 0.0)
                scr[b, :, pl.ds(t * cm, cm)] = y
                s1 = jnp.sum(y, axis=1, keepdims=True)
                s2 = jnp.sum(y * y, axis=1, keepdims=True)
                ssum = s1 if ssum is None else ssum + s1
                ssq = s2 if ssq is None else ssq + s2
        mean = ssum / cnt
        var = ssq / cnt - mean * mean
        inv = lax.rsqrt(var + 1e-5)
        gam = g_ref[...]
        bet = bt_ref[...]
        for b in range(_B):
            for t in range(T2):
                y = (scr[b, :, pl.ds(t * cm, cm)] - mean) * inv * gam + bet
                y = jnp.maximum(y, 0.0)
                if final:
                    y = _dot(y, lw_ref[...]) + lb_ref[...]
                    o_ref[b, :, pl.ds(t, 1)] = y
                else:
                    o_ref[b, :, pl.ds(t * cm, cm)] = y

    return pl.pallas_call(
        body,
        compiler_params=pltpu.CompilerParams(vmem_limit_bytes=100 * 1024 * 1024),
        grid=(_NP // nblk,),
        in_specs=[
            pl.BlockSpec((_B, nblk, T * cm), lambda i: (0, i, 0)),
            pl.BlockSpec((3 * cm, 3 * cm), lambda i: (0, 0)),
            pl.BlockSpec((1, 3 * cm), lambda i: (0, 0)),
            pl.BlockSpec((nblk, 1), lambda i: (i, 0)),
            pl.BlockSpec((nblk, 1), lambda i: (i, 0)),
            pl.BlockSpec((cm, 1), lambda i: (0, 0)),
            pl.BlockSpec((1, 1), lambda i: (0, 0)),
        ],
        out_specs=pl.BlockSpec((_B, nblk, T2 if final else T2 * cm),
                               lambda i: (0, i, 0)),
        out_shape=jax.ShapeDtypeStruct(
            (_B, _NP, T2 if final else T2 * cm), jnp.float32),
        scratch_shapes=[pltpu.VMEM((_B, nblk, T2 * cm), jnp.float32)],
    )


def _tc_weights(p, cin, cout):
    ws = []
    bs = []
    for i in range(3):
        w = p['w%d' % (i + 1)]
        ws.append(jnp.transpose(w[:, :, 0, :], (2, 1, 0)).reshape(3 * cin, cout))
        bs.append(p['b%d' % (i + 1)])
    return jnp.concatenate(ws, axis=1), jnp.concatenate(bs).reshape(1, 3 * cout)


def _stconv(h, rowp, colp, nwp, p, cin, final, lin_w, lin_b):
    T = h.shape[2] // cin
    T1 = T - 2
    w1, b1 = _tc_weights(p['tc1'], cin, _HID)
    t0 = _make_tconv(T, cin, _HID, 1024)(h, w1, b1)
    J = _B * T1
    V0 = jnp.transpose(t0, (1, 0, 2)).reshape(_NP, J * _HID)
    s1 = _prop_all(V0, rowp, colp, nwp)
    s2 = _prop_all(s1, rowp, colp, nwp)
    M = _NP * J
    cheb_w = p['cheb_w'].reshape(3 * _HID, _HID)
    cheb_b = p['cheb_b'].reshape(1, _HID)
    g = _make_cheb_combine(M, 2048)(
        V0.reshape(M, _HID), s1.reshape(M, _HID), s2.reshape(M, _HID),
        cheb_w, cheb_b)
    u = jnp.transpose(g.reshape(_NP, _B, T1 * _HID), (1, 0, 2))
    w2, b2 = _tc_weights(p['tc2'], _HID, _HID)
    npad = _NP - _N
    g2 = jnp.concatenate([p['bn_g'], jnp.ones((npad,), jnp.float32)]).reshape(_NP, 1)
    bt2 = jnp.concatenate([p['bn_b'], jnp.zeros((npad,), jnp.float32)]).reshape(_NP, 1)
    return _make_tconv_bn(T1, 1024, final)(u, w2, b2, g2, bt2, lin_w, lin_b)


def kernel(x, edge_index, edge_weight, params):
    row = edge_index[0]
    col = edge_index[1]
    ew = jnp.where(row == col, 0.0, edge_weight)
    pad = _EPAD - row.shape[0]
    zi = jnp.zeros((pad,), jnp.int32)
    rowp = jnp.concatenate([row, zi])
    colp = jnp.concatenate([col, zi])
    ewp = jnp.concatenate([ew, jnp.zeros((pad,), jnp.float32)])

    ones = jnp.ones((_NP, 256), jnp.float32)
    dega = _make_prop(1)(ones, jnp.zeros((_NP, 128), jnp.float32),
                         rowp, rowp,
                         jnp.broadcast_to(ewp[:, None], (_EPAD, 16))
                         .reshape(_EPAD // 8, 128))
    deg = dega[:, 0]
    dis = jnp.where(deg > 0, lax.rsqrt(jnp.where(deg > 0, deg, 1.0)), 0.0)
    nwp = _make_nw()(dis, rowp, colp, ewp)

    B, T, _, cin = x.shape
    xn = jnp.transpose(x, (0, 2, 1, 3)).reshape(B, _N, T * cin)
    xn = jnp.concatenate(
        [xn, jnp.zeros((B, _NP - _N, T * cin), jnp.float32)], axis=1)
    lin_w = params['lin_w'].T
    lin_b = params['lin_b'].reshape(1, 1)
    h = _stconv(xn, rowp, colp, nwp, params['stconv1'], 2, False, lin_w, lin_b)
    h = _stconv(h, rowp, colp, nwp, params['stconv2'], _HID, True, lin_w, lin_b)
    return jnp.transpose(h[:, :_N, :], (0, 2, 1))[..., None]


# speedup vs baseline: 1.1559x; 1.0757x over previous
import functools

import jax
import jax.numpy as jnp
from jax import lax
from jax.experimental import pallas as pl
from jax.experimental.pallas import tpu as pltpu
from jax.experimental.pallas import tpu_sc as plsc

_N = 10000
_NP = 10240
_B = 2
_EPAD = 163840
_BLK = 64
_HID = 32


@functools.lru_cache(maxsize=None)
def _make_prop(nc):
    mesh = plsc.VectorSubcoreMesh(core_axis_name="c", subcore_axis_name="s")
    ept = _EPAD // 16
    nblk = ept // _BLK
    npt = _NP // 16
    cc = 128
    ns = 4

    @functools.partial(
        pl.kernel, mesh=mesh,
        out_type=jax.ShapeDtypeStruct((_NP, 2 * nc * cc), jnp.float32),
        scratch_types=(
            [pltpu.VMEM((_BLK,), jnp.int32) for _ in range(ns)]
            + [pltpu.VMEM((_BLK,), jnp.int32) for _ in range(ns)]
            + [pltpu.VMEM((_BLK // 8, 128), jnp.float32) for _ in range(ns)]
            + [pltpu.VMEM((_BLK, cc), jnp.float32) for _ in range(ns)]
            + [pltpu.VMEM_SHARED((_NP, cc), jnp.float32)]
            + [pltpu.SemaphoreType.DMA for _ in range(4 * ns)]
        ),
    )
    def prop(vh, z, rowh, colh, wh, oh, *bufs):
        rv = bufs[0:ns]
        cv = bufs[ns:2 * ns]
        wv = bufs[2 * ns:3 * ns]
        gb = bufs[3 * ns:4 * ns]
        acc = bufs[4 * ns]
        sr = bufs[4 * ns + 1:4 * ns + 1 + ns]
        scw = bufs[4 * ns + 1 + ns:4 * ns + 1 + 2 * ns]
        sg = bufs[4 * ns + 1 + 2 * ns:4 * ns + 1 + 3 * ns]
        ss = bufs[4 * ns + 1 + 3 * ns:4 * ns + 1 + 4 * ns]
        c = lax.axis_index("c")
        s = lax.axis_index("s")
        rstripe = pl.ds(s * npt, npt)
        hb = pl.ds(0, _BLK)
        hw = pl.ds(0, _BLK // 8)

        def row_start(i, h):
            pltpu.async_copy(rowh.at[pl.ds(s * ept + i * _BLK, _BLK)],
                             rv[h], sr[h])

        def row_drain(h):
            pltpu.make_async_copy(rowh.at[hb], rv[h], sr[h]).wait()

        def colw_start(i, h):
            pltpu.async_copy(colh.at[pl.ds(s * ept + i * _BLK, _BLK)],
                             cv[h], scw[h])
            woff = pl.multiple_of((s * ept + i * _BLK) // 8, 8)
            pltpu.async_copy(wh.at[pl.ds(woff, _BLK // 8), :], wv[h], scw[h])

        def colw_drain(h):
            pltpu.make_async_copy(colh.at[hb], cv[h], scw[h]).wait()
            pltpu.make_async_copy(wh.at[hw, :], wv[h], scw[h]).wait()

        def scat_drain(h):
            pltpu.make_async_copy(z.at[hb], gb[h], ss[h]).wait()

        for k in range(nc):
            off = pl.multiple_of((c * nc + k) * cc, cc)
            csl = pl.ds(off, cc)
            pltpu.sync_copy(z.at[rstripe], acc.at[rstripe])
            plsc.subcore_barrier()

            for j in range(ns):
                row_start(j, j)
            for j in range(2):
                colw_start(j, j)
            for j in range(2):
                row_drain(j)
                pltpu.async_copy(vh.at[rv[j], csl], gb[j], sg[j])

            def body(i4, carry):
                for h in range(ns):
                    h2 = (h + 2) % ns
                    i = i4 * ns + h
                    pltpu.make_async_copy(vh.at[rv[h], csl], gb[h], sg[h]).wait()

                    @pl.when(i + 4 < nblk)
                    def _():
                        row_start(i + 4, h)

                    @pl.when(i + 2 < nblk)
                    def _():
                        @pl.when(i >= 2)
                        def _():
                            scat_drain(h2)
                        row_drain(h2)
                        pltpu.async_copy(vh.at[rv[h2], csl], gb[h2], sg[h2])
                        colw_start(i + 2, h2)

                    colw_drain(h)
                    def sbody(g, cy):
                        for el in range(8):
                            wvec = wv[h][g, pl.ds(el * 16, 16)]
                            e = g * 8 + el
                            for j in range(cc // 16):
                                sl = pl.ds(j * 16, 16)
                                gb[h][e, sl] = gb[h][e, sl] * wvec
                        return cy
                    lax.fori_loop(0, _BLK // 8, sbody, 0, unroll=2)

                    pltpu.async_copy(gb[h], acc.at[cv[h]], ss[h], add=True)
                return carry

            lax.fori_loop(0, nblk // ns, body, 0)
            scat_drain((nblk - 2) % ns)
            scat_drain((nblk - 1) % ns)
            plsc.subcore_barrier()
            pltpu.sync_copy(acc.at[rstripe], oh.at[rstripe, csl])
            plsc.subcore_barrier()

    return prop


@functools.lru_cache(maxsize=None)
def _make_deg():
    mesh = plsc.VectorSubcoreMesh(core_axis_name="c", subcore_axis_name="s")
    epw = _EPAD // 32
    npt = _NP // 16

    @functools.partial(
        pl.kernel, mesh=mesh,
        out_type=jax.ShapeDtypeStruct((2, _NP), jnp.float32),
        compiler_params=pltpu.CompilerParams(needs_layout_passes=False),
        scratch_types=[
            pltpu.VMEM((_NP,), jnp.float32),
            pltpu.VMEM((epw,), jnp.int32),
            pltpu.VMEM((epw,), jnp.float32),
            pltpu.VMEM((npt,), jnp.float32),
            pltpu.VMEM((npt,), jnp.float32),
            pltpu.VMEM_SHARED((16, _NP), jnp.float32),
        ],
    )
    def degk(zh, rowh, wh, oh, degv, rowv, wv, tacc, tin, hist):
        c = lax.axis_index("c")
        s = lax.axis_index("s")
        base = (s * 2 + c) * epw
        pltpu.sync_copy(rowh.at[pl.ds(base, epw)], rowv)
        pltpu.sync_copy(wh.at[pl.ds(base, epw)], wv)
        pltpu.sync_copy(zh, degv)

        def body(i, cy):
            sl = pl.ds(i * 16, 16)
            plsc.addupdate_scatter(degv, [rowv[sl]], wv[sl])
            return cy

        lax.fori_loop(0, epw // 16, body, 0)
        pltpu.sync_copy(degv, hist.at[s])
        plsc.subcore_barrier()

        stripe = pl.ds(s * npt, npt)
        pltpu.sync_copy(hist.at[0, stripe], tacc)
        for t in range(1, 16):
            pltpu.sync_copy(hist.at[t, stripe], tin)

            def abody(i, cy):
                sl = pl.ds(i * 16, 16)
                tacc[sl] = tacc[sl] + tin[sl]
                return cy

            lax.fori_loop(0, npt // 16, abody, 0)
        pltpu.sync_copy(tacc, oh.at[c, stripe])

    return degk


@functools.lru_cache(maxsize=None)
def _make_nw():
    mesh = plsc.VectorSubcoreMesh(core_axis_name="c", subcore_axis_name="s")
    epw = _EPAD // 32

    @functools.partial(
        pl.kernel, mesh=mesh,
        out_type=jax.ShapeDtypeStruct((_EPAD,), jnp.float32),
        compiler_params=pltpu.CompilerParams(needs_layout_passes=False),
        scratch_types=[
            pltpu.VMEM((_NP,), jnp.float32),
            pltpu.VMEM((epw,), jnp.int32),
            pltpu.VMEM((epw,), jnp.int32),
            pltpu.VMEM((epw,), jnp.float32),
            pltpu.VMEM((epw,), jnp.float32),
        ],
    )
    def nwk(dish, rowh, colh, ewh, nwh, disv, rowv, colv, ewv, nwv):
        c = lax.axis_index("c")
        s = lax.axis_index("s")
        base = (s * 2 + c) * epw
        pltpu.sync_copy(dish, disv)
        pltpu.sync_copy(rowh.at[pl.ds(base, epw)], rowv)
        pltpu.sync_copy(colh.at[pl.ds(base, epw)], colv)
        pltpu.sync_copy(ewh.at[pl.ds(base, epw)], ewv)

        def body(i, cy):
            sl = pl.ds(i * 16, 16)
            r16 = rowv[sl]
            c16 = colv[sl]
            e16 = ewv[sl]
            dr = plsc.load_gather(disv, [r16])
            dc = plsc.load_gather(disv, [c16])
            v = -(dr * e16 * dc)
            v = jnp.where(r16 == c16, jnp.zeros_like(v), v)
            nwv[sl] = v
            return cy

        lax.fori_loop(0, epw // 16, body, 0)
        pltpu.sync_copy(nwv, nwh.at[pl.ds(base, epw)])

    return nwk


def _prop_all(V, rowp, colp, w):
    C = V.shape[1]
    nc = -(-C // 256)
    cpad = nc * 256
    if cpad != C:
        V = jnp.concatenate([V, jnp.zeros((_NP, cpad - C), jnp.float32)], axis=1)
    z = jnp.zeros((_NP, 128), jnp.float32)
    w16 = jnp.broadcast_to(w[:, None], (_EPAD, 16)).reshape(_EPAD // 8, 128)
    out = _make_prop(nc)(V, z, rowp, colp, w16)
    return out[:, :C] if cpad != C else out


def _dot(a, b):
    return lax.dot_general(a, b, (((1,), (0,)), ((), ())),
                           preferred_element_type=jnp.float32)


@functools.lru_cache(maxsize=None)
def _make_tconv(T, cin, cout, nblk):
    T1 = T - 2

    def body(x_ref, w_ref, b_ref, o_ref):
        w = w_ref[...]
        bb = b_ref[...]
        for t in range(T1):
            xs = x_ref[0, :, pl.ds(t * cin, 3 * cin)]
            h = _dot(xs, w) + bb
            p = h[:, :cout]
            q = h[:, cout:2 * cout]
            r = h[:, 2 * cout:]
            o_ref[0, :, pl.ds(t * cout, cout)] = jnp.maximum(
                p * jax.nn.sigmoid(q) + r, 0.0)

    return pl.pallas_call(
        body,
        compiler_params=pltpu.CompilerParams(vmem_limit_bytes=100 * 1024 * 1024),
        grid=(_NP // nblk, _B),
        in_specs=[
            pl.BlockSpec((1, nblk, T * cin), lambda i, b: (b, i, 0)),
            pl.BlockSpec((3 * cin, 3 * cout), lambda i, b: (0, 0)),
            pl.BlockSpec((1, 3 * cout), lambda i, b: (0, 0)),
        ],
        out_specs=pl.BlockSpec((1, nblk, T1 * cout), lambda i, b: (b, i, 0)),
        out_shape=jax.ShapeDtypeStruct((_B, _NP, T1 * cout), jnp.float32),
    )


@functools.lru_cache(maxsize=None)
def _make_cheb_combine(M, mb):
    def body(t0_ref, s1_ref, s2_ref, w_ref, b_ref, o_ref):
        t0 = t0_ref[...]
        s1 = s1_ref[...]
        s2 = s2_ref[...]
        w = w_ref[...]
        y = (_dot(t0, w[0:32]) + _dot(s1, w[32:64])
             + _dot(2.0 * s2 - t0, w[64:96]) + b_ref[...])
        o_ref[...] = jnp.maximum(y, 0.0)

    return pl.pallas_call(
        body,
        compiler_params=pltpu.CompilerParams(vmem_limit_bytes=100 * 1024 * 1024),
        grid=(M // mb,),
        in_specs=[
            pl.BlockSpec((mb, 32), lambda i: (i, 0)),
            pl.BlockSpec((mb, 32), lambda i: (i, 0)),
            pl.BlockSpec((mb, 32), lambda i: (i, 0)),
            pl.BlockSpec((96, 32), lambda i: (0, 0)),
            pl.BlockSpec((1, 32), lambda i: (0, 0)),
        ],
        out_specs=pl.BlockSpec((mb, 32), lambda i: (i, 0)),
        out_shape=jax.ShapeDtypeStruct((M, 32), jnp.float32),
    )


@functools.lru_cache(maxsize=None)
def _make_tconv_bn(T, nblk, final):
    T2 = T - 2
    cm = _HID
    cnt = float(_B * T2 * cm)

    def body(u_ref, w_ref, b_ref, g_ref, bt_ref, lw_ref, lb_ref, o_ref, scr):
        w = w_ref[...]
        bb = b_ref[...]
        ssum = None
        ssq = None
        for b in range(_B):
            for t in range(T2):
                xs = u_ref[b, :, pl.ds(t * cm, 3 * cm)]
                h = _dot(xs, w) + bb
                p = h[:, :cm]
                q = h[:, cm:2 * cm]
                r = h[:, 2 * cm:]
                y = jnp.maximum(p * jax.nn.sigmoid(q) + r, 0.0)
                scr[b, :, pl.ds(t * cm, cm)] = y
                s1 = jnp.sum(y, axis=1, keepdims=True)
                s2 = jnp.sum(y * y, axis=1, keepdims=True)
                ssum = s1 if ssum is None else ssum + s1
                ssq = s2 if ssq is None else ssq + s2
        mean = ssum / cnt
        var = ssq / cnt - mean * mean
        inv = lax.rsqrt(var + 1e-5)
        gam = g_ref[...]
        bet = bt_ref[...]
        for b in range(_B):
            for t in range(T2):
                y = (scr[b, :, pl.ds(t * cm, cm)] - mean) * inv * gam + bet
                y = jnp.maximum(y, 0.0)
                if final:
                    y = _dot(y, lw_ref[...]) + lb_ref[...]
                    o_ref[b, :, pl.ds(t, 1)] = y
                else:
                    o_ref[b, :, pl.ds(t * cm, cm)] = y

    return pl.pallas_call(
        body,
        compiler_params=pltpu.CompilerParams(vmem_limit_bytes=100 * 1024 * 1024),
        grid=(_NP // nblk,),
        in_specs=[
            pl.BlockSpec((_B, nblk, T * cm), lambda i: (0, i, 0)),
            pl.BlockSpec((3 * cm, 3 * cm), lambda i: (0, 0)),
            pl.BlockSpec((1, 3 * cm), lambda i: (0, 0)),
            pl.BlockSpec((nblk, 1), lambda i: (i, 0)),
            pl.BlockSpec((nblk, 1), lambda i: (i, 0)),
            pl.BlockSpec((cm, 1), lambda i: (0, 0)),
            pl.BlockSpec((1, 1), lambda i: (0, 0)),
        ],
        out_specs=pl.BlockSpec((_B, nblk, T2 if final else T2 * cm),
                               lambda i: (0, i, 0)),
        out_shape=jax.ShapeDtypeStruct(
            (_B, _NP, T2 if final else T2 * cm), jnp.float32),
        scratch_shapes=[pltpu.VMEM((_B, nblk, T2 * cm), jnp.float32)],
    )


def _tc_weights(p, cin, cout):
    ws = []
    bs = []
    for i in range(3):
        w = p['w%d' % (i + 1)]
        ws.append(jnp.transpose(w[:, :, 0, :], (2, 1, 0)).reshape(3 * cin, cout))
        bs.append(p['b%d' % (i + 1)])
    return jnp.concatenate(ws, axis=1), jnp.concatenate(bs).reshape(1, 3 * cout)


def _stconv(h, rowp, colp, nwp, p, cin, final, lin_w, lin_b):
    T = h.shape[2] // cin
    T1 = T - 2
    w1, b1 = _tc_weights(p['tc1'], cin, _HID)
    t0 = _make_tconv(T, cin, _HID, 1024)(h, w1, b1)
    J = _B * T1
    V0 = jnp.transpose(t0, (1, 0, 2)).reshape(_NP, J * _HID)
    s1 = _prop_all(V0, rowp, colp, nwp)
    s2 = _prop_all(s1, rowp, colp, nwp)
    M = _NP * J
    cheb_w = p['cheb_w'].reshape(3 * _HID, _HID)
    cheb_b = p['cheb_b'].reshape(1, _HID)
    g = _make_cheb_combine(M, 2048)(
        V0.reshape(M, _HID), s1.reshape(M, _HID), s2.reshape(M, _HID),
        cheb_w, cheb_b)
    u = jnp.transpose(g.reshape(_NP, _B, T1 * _HID), (1, 0, 2))
    w2, b2 = _tc_weights(p['tc2'], _HID, _HID)
    npad = _NP - _N
    g2 = jnp.concatenate([p['bn_g'], jnp.ones((npad,), jnp.float32)]).reshape(_NP, 1)
    bt2 = jnp.concatenate([p['bn_b'], jnp.zeros((npad,), jnp.float32)]).reshape(_NP, 1)
    return _make_tconv_bn(T1, 1024, final)(u, w2, b2, g2, bt2, lin_w, lin_b)


def kernel(x, edge_index, edge_weight, params):
    row = edge_index[0]
    col = edge_index[1]
    ew = jnp.where(row == col, 0.0, edge_weight)
    pad = _EPAD - row.shape[0]
    zi = jnp.zeros((pad,), jnp.int32)
    rowp = jnp.concatenate([row, zi])
    colp = jnp.concatenate([col, zi])
    ewp = jnp.concatenate([ew, jnp.zeros((pad,), jnp.float32)])

    degp = _make_deg()(jnp.zeros((_NP,), jnp.float32), rowp, ewp)
    deg = degp[0] + degp[1]
    dis = jnp.where(deg > 0, lax.rsqrt(jnp.where(deg > 0, deg, 1.0)), 0.0)
    nwp = _make_nw()(dis, rowp, colp, ewp)

    B, T, _, cin = x.shape
    xn = jnp.transpose(x, (0, 2, 1, 3)).reshape(B, _N, T * cin)
    xn = jnp.concatenate(
        [xn, jnp.zeros((B, _NP - _N, T * cin), jnp.float32)], axis=1)
    lin_w = params['lin_w'].T
    lin_b = params['lin_b'].reshape(1, 1)
    h = _stconv(xn, rowp, colp, nwp, params['stconv1'], 2, False, lin_w, lin_b)
    h = _stconv(h, rowp, colp, nwp, params['stconv2'], _HID, True, lin_w, lin_b)
    return jnp.transpose(h[:, :_N, :], (0, 2, 1))[..., None]
